# Initial kernel scaffold; baseline (speedup 1.0000x reference)
import jax
import jax.numpy as jnp
from jax.experimental import pallas as pl


def kernel(x, edge_index, edge_attr, batch, W_edge, W1, b1, W2, b2, eps):
    raise NotImplementedError("write your pallas kernel here")


# speedup vs baseline: 2.9472x; 2.9472x over previous
import functools

import jax
import jax.numpy as jnp
from jax import lax
from jax.experimental import pallas as pl
from jax.experimental.pallas import tpu as pltpu
from jax.experimental.pallas import tpu_sc as plsc

N = 10000
E = 320000
D = 128
DE = 16
NLAYER = 3
G = 512

NC, NS = 2, 16
NW = NC * NS
CH = 128
NCHUNK = E // CH
CPW = NCHUNK // NW
EXTRA = NCHUNK - CPW * NW
RPT = 624
LAST_RPT = N - RPT * (NS - 1)

NB = 1000
NBLK = N // NB
BE = 4000


def _eemb_body(ea_ref, we_ref, e0_ref, e1_ref, e2_ref):
    ea = ea_ref[...]
    e0_ref[...] = jnp.dot(ea, we_ref[0], preferred_element_type=jnp.float32)
    e1_ref[...] = jnp.dot(ea, we_ref[1], preferred_element_type=jnp.float32)
    e2_ref[...] = jnp.dot(ea, we_ref[2], preferred_element_type=jnp.float32)


def _eemb(edge_attr, w_edge):
    out = jax.ShapeDtypeStruct((E, D), jnp.float32)
    return pl.pallas_call(
        _eemb_body,
        grid=(E // BE,),
        in_specs=[
            pl.BlockSpec((BE, DE), lambda i: (i, 0)),
            pl.BlockSpec((NLAYER, DE, D), lambda i: (0, 0, 0)),
        ],
        out_specs=[
            pl.BlockSpec((BE, D), lambda i: (i, 0)),
            pl.BlockSpec((BE, D), lambda i: (i, 0)),
            pl.BlockSpec((BE, D), lambda i: (i, 0)),
        ],
        out_shape=[out, out, out],
    )(edge_attr, w_edge)


def _make_sc_msg():
    mesh = plsc.VectorSubcoreMesh(
        core_axis_name="c", subcore_axis_name="s", num_cores=NC, num_subcores=NS
    )

    @functools.partial(
        pl.kernel,
        out_type=jax.ShapeDtypeStruct((NC, N, D), jnp.float32),
        mesh=mesh,
        scratch_types=[
            pltpu.VMEM_SHARED((N, D), jnp.float32),
            pltpu.VMEM((CH,), jnp.int32),
            pltpu.VMEM((CH,), jnp.int32),
            pltpu.VMEM((CH, D), jnp.float32),
            pltpu.VMEM((CH, D), jnp.float32),
            pltpu.SemaphoreType.DMA,
        ],
    )
    def sc_msg(h_hbm, e_hbm, src_hbm, dst_hbm, zeros_hbm, out_hbm,
               agg, src_v, dst_v, rows_v, emb_v, gsem):
        cid = lax.axis_index("c")
        sid = lax.axis_index("s")
        wid = sid * NC + cid
        r0 = sid * RPT

        pltpu.sync_copy(zeros_hbm.at[pl.ds(r0, RPT), :],
                        agg.at[pl.ds(r0, RPT), :])

        @pl.when(sid == NS - 1)
        def _():
            pltpu.sync_copy(zeros_hbm.at[pl.ds(RPT * NS, N - RPT * NS), :],
                            agg.at[pl.ds(RPT * NS, N - RPT * NS), :])

        plsc.subcore_barrier()

        def process(j):
            pltpu.sync_copy(src_hbm.at[j], src_v)
            pltpu.sync_copy(dst_hbm.at[j], dst_v)
            pltpu.async_copy(h_hbm.at[src_v], rows_v, gsem).wait()
            pltpu.sync_copy(e_hbm.at[pl.ds(j * CH, CH), :], emb_v)

            def row_body(r, c):
                for q in range(D // 16):
                    sl = pl.ds(q * 16, 16)
                    rows_v[r, sl] = jnp.maximum(rows_v[r, sl] + emb_v[r, sl],
                                                0.0)
                return c

            lax.fori_loop(0, CH, row_body, 0)
            pltpu.sync_copy(rows_v, agg.at[dst_v], add=True)

        def chunk_body(c, carry):
            process(wid * CPW + c)
            return carry

        lax.fori_loop(0, CPW, chunk_body, 0)

        @pl.when(wid < EXTRA)
        def _():
            process(NW * CPW + wid)

        plsc.subcore_barrier()
        pltpu.sync_copy(agg.at[pl.ds(r0, RPT), :],
                        out_hbm.at[cid, pl.ds(r0, RPT), :])

        @pl.when(sid == NS - 1)
        def _():
            pltpu.sync_copy(agg.at[pl.ds(RPT * NS, N - RPT * NS), :],
                            out_hbm.at[cid, pl.ds(RPT * NS, N - RPT * NS), :])

    return sc_msg


def _mlp_body(last, parts_ref, h_ref, w1_ref, b1_ref, w2_ref, b2_ref,
              scale_ref, out_ref):
    t = parts_ref[0] + parts_ref[1] + scale_ref[0, 0] * h_ref[...]
    u = jnp.dot(t, w1_ref[...], preferred_element_type=jnp.float32)
    u = jnp.maximum(u + b1_ref[...], 0.0)
    v = jnp.dot(u, w2_ref[...], preferred_element_type=jnp.float32)
    v = v + b2_ref[...]
    if not last:
        v = jnp.maximum(v, 0.0)
    out_ref[...] = v


def _mlp(parts, h, w1, b1, w2, b2, scale, last):
    return pl.pallas_call(
        functools.partial(_mlp_body, last),
        grid=(NBLK,),
        in_specs=[
            pl.BlockSpec((NC, NB, D), lambda i: (0, i, 0)),
            pl.BlockSpec((NB, D), lambda i: (i, 0)),
            pl.BlockSpec((D, D), lambda i: (0, 0)),
            pl.BlockSpec((1, D), lambda i: (0, 0)),
            pl.BlockSpec((D, D), lambda i: (0, 0)),
            pl.BlockSpec((1, D), lambda i: (0, 0)),
            pl.BlockSpec((1, 1), lambda i: (0, 0)),
        ],
        out_specs=pl.BlockSpec((NB, D), lambda i: (i, 0)),
        out_shape=jax.ShapeDtypeStruct((N, D), jnp.float32),
    )(parts, h, w1, b1, w2, b2, scale)


def _pool_body(batch_ref, h_ref, out_ref, sums, counts):
    i = pl.program_id(0)
    b = batch_ref[0, 0, :]
    gid = lax.broadcasted_iota(jnp.int32, (G, NB), 0)
    mask = (b[None, :] == gid).astype(jnp.float32)
    psum = jnp.dot(mask, h_ref[...], preferred_element_type=jnp.float32)
    pcnt = jnp.broadcast_to(jnp.sum(mask, axis=1, keepdims=True), (G, D))

    @pl.when(i == 0)
    def _():
        sums[...] = psum
        counts[...] = pcnt

    @pl.when(i > 0)
    def _():
        sums[...] += psum
        counts[...] += pcnt

    @pl.when(i == NBLK - 1)
    def _():
        out_ref[...] = sums[...] / jnp.maximum(counts[...], 1.0)


def _pool(batch3d, h):
    return pl.pallas_call(
        _pool_body,
        grid=(NBLK,),
        in_specs=[
            pl.BlockSpec((1, 1, NB), lambda i: (i, 0, 0)),
            pl.BlockSpec((NB, D), lambda i: (i, 0)),
        ],
        out_specs=pl.BlockSpec((G, D), lambda i: (0, 0)),
        out_shape=jax.ShapeDtypeStruct((G, D), jnp.float32),
        scratch_shapes=[
            pltpu.VMEM((G, D), jnp.float32),
            pltpu.VMEM((G, D), jnp.float32),
        ],
    )(batch3d, h)


def kernel(x, edge_index, edge_attr, batch, W_edge, W1, b1, W2, b2, eps):
    src = edge_index[0].astype(jnp.int32).reshape(NCHUNK, CH)
    dst = edge_index[1].astype(jnp.int32).reshape(NCHUNK, CH)
    batch3d = batch.astype(jnp.int32).reshape(NBLK, 1, NB)
    zeros = jnp.zeros((N, D), jnp.float32)

    e_embs = _eemb(edge_attr, W_edge)
    sc_msg = _make_sc_msg()

    h = x
    for l in range(NLAYER):
        parts = sc_msg(h, e_embs[l], src, dst, zeros)
        scale = (1.0 + eps[l]).reshape(1, 1)
        h = _mlp(parts, h, W1[l], b1[l].reshape(1, D), W2[l],
                 b2[l].reshape(1, D), scale, last=(l == NLAYER - 1))
    return _pool(batch3d, h)
